# baseline (speedup 1.0000x reference)
import functools

import jax
import jax.numpy as jnp
from jax.experimental import pallas as pl

TINY_NUMBER = 1e-6
N_RAY, N_SAMPLE, N_SRC, DIM = 512, 64, 8, 64
H = DIM // 8

BLK_T = 256


def _body(q_ref, k_ref, pos_ref, m_ref,
          wq_ref, wk_ref, wv_ref, wp1_ref, bp1_ref, wp2_ref, bp2_ref,
          wa1_ref, ba1_ref, wa2_ref, ba2_ref, wo_ref, bo_ref,
          x_ref, attn_ref, kstd_ref, nkstd_ref):
    T = q_ref.shape[0]
    N = N_SRC
    TN = T * N

    f32 = jnp.float32
    dot = functools.partial(jnp.dot, preferred_element_type=f32)

    q = q_ref[...]
    k = k_ref[...]
    pos = pos_ref[...]
    m = m_ref[...]

    qf = dot(q, wq_ref[...])
    kf = dot(k, wk_ref[...])
    vf = dot(k, wv_ref[...])

    ph = jnp.maximum(dot(pos, wp1_ref[...]) + bp1_ref[...], 0.0)
    posf = dot(ph, wp2_ref[...]) + bp2_ref[...]

    m3 = m.reshape(T, N, 1)
    cnt = jnp.sum(m3, axis=1, keepdims=True)
    w3 = jnp.where(cnt == 0.0, 1.0, m3)
    cnt_eff = jnp.sum(w3, axis=1, keepdims=True)
    inv_cnt = 1.0 / cnt_eff

    kf3 = kf.reshape(T, N, DIM)
    mean_k = jnp.sum(kf3 * w3, axis=1, keepdims=True) * inv_cnt
    var = jnp.sum(((kf3 - mean_k) ** 2) * w3, axis=1, keepdims=True) \
        / jnp.maximum(cnt_eff - 1.0, 1.0)
    single = cnt_eff == 1.0
    std = jnp.sqrt(jnp.where(single, 1.0, jnp.maximum(var, 0.0)))
    mean_abs = jnp.sum(jnp.abs(kf3) * w3, axis=1, keepdims=True) * inv_cnt
    kstd = jnp.where(single, 0.0, std)
    nkstd = jnp.where(single, 0.0, std / (mean_abs + TINY_NUMBER))
    kstd_ref[...] = kstd.reshape(T, DIM)
    nkstd_ref[...] = nkstd.reshape(T, DIM)

    posf3 = posf.reshape(T, N, DIM)
    ap = (kf3 - qf[:, None, :] + posf3).reshape(TN, DIM)
    a1 = jnp.maximum(dot(ap, wa1_ref[...]) + ba1_ref[...], 0.0)
    logits = (dot(a1, wa2_ref[...]) + ba2_ref[...]).reshape(T, N, DIM)

    valid = w3 != 0.0
    lmax = jnp.max(jnp.where(valid, logits, -jnp.inf), axis=1, keepdims=True)
    e = jnp.where(valid, jnp.exp(logits - lmax), 0.0)
    attn = e / jnp.sum(e, axis=1, keepdims=True)
    attn_ref[...] = attn.reshape(TN, DIM)

    x = jnp.sum((vf.reshape(T, N, DIM) + posf3) * attn, axis=1)
    x_ref[...] = dot(x, wo_ref[...]) + bo_ref[...]


def kernel(q, k, pos, mask, Wq, Wk, Wv, Wp1, bp1, Wp2, bp2, Wa1, ba1, Wa2, ba2, Wo, bo):
    R, S, N, D = N_RAY, N_SAMPLE, N_SRC, DIM
    RS = R * S
    q2 = q.reshape(RS, D)
    k2 = k.reshape(RS * N, D)
    pos2 = pos.reshape(RS * N, 4)
    m2 = mask.reshape(RS * N, 1).astype(jnp.float32)

    grid = (RS // BLK_T,)
    tok = lambda i: (i, 0)
    rep = lambda i: (0, 0)

    def wspec(w):
        return pl.BlockSpec(w.shape, rep)

    bp1r, bp2r = bp1.reshape(1, H), bp2.reshape(1, D)
    ba1r, ba2r = ba1.reshape(1, H), ba2.reshape(1, D)
    bor = bo.reshape(1, D)

    out = pl.pallas_call(
        _body,
        grid=grid,
        in_specs=[
            pl.BlockSpec((BLK_T, D), tok),
            pl.BlockSpec((BLK_T * N, D), tok),
            pl.BlockSpec((BLK_T * N, 4), tok),
            pl.BlockSpec((BLK_T * N, 1), tok),
            wspec(Wq), wspec(Wk), wspec(Wv),
            wspec(Wp1), wspec(bp1r), wspec(Wp2), wspec(bp2r),
            wspec(Wa1), wspec(ba1r), wspec(Wa2), wspec(ba2r),
            wspec(Wo), wspec(bor),
        ],
        out_specs=[
            pl.BlockSpec((BLK_T, D), tok),
            pl.BlockSpec((BLK_T * N, D), tok),
            pl.BlockSpec((BLK_T, D), tok),
            pl.BlockSpec((BLK_T, D), tok),
        ],
        out_shape=[
            jax.ShapeDtypeStruct((RS, D), jnp.float32),
            jax.ShapeDtypeStruct((RS * N, D), jnp.float32),
            jax.ShapeDtypeStruct((RS, D), jnp.float32),
            jax.ShapeDtypeStruct((RS, D), jnp.float32),
        ],
    )(q2, k2, pos2, m2, Wq, Wk, Wv, Wp1, bp1r, Wp2, bp2r,
      Wa1, ba1r, Wa2, ba2r, Wo, bor)

    x, attn, kstd, nkstd = out
    return (x.reshape(R, S, D), attn.reshape(R, S, N, D),
            kstd.reshape(R, S, D), nkstd.reshape(R, S, D))


# speedup vs baseline: 1.4900x; 1.4900x over previous
import functools

import jax
import jax.numpy as jnp
from jax.experimental import pallas as pl

TINY_NUMBER = 1e-6
N_RAY, N_SAMPLE, N_SRC, DIM = 512, 64, 8, 64
H = DIM // 8
NPAIR = N_SRC // 2
PW = 2 * DIM

BLK_T = 256


def _roll64(a):
    return jnp.concatenate([a[:, DIM:], a[:, :DIM]], axis=1)


def _body(q_ref, k_ref, pos_ref, m_ref,
          wq2_ref, wk2_ref, wv2_ref, wp1b_ref, bp1a_ref, wp2c_ref, bp2p_ref,
          wa12_ref, ba1p_ref, wa22_ref, ba2p_ref, wos_ref, bo_ref, esel_ref,
          x_ref, attn_ref, kstd_ref, nkstd_ref):
    T = q_ref.shape[0]
    f32 = jnp.float32
    dot = functools.partial(jnp.dot, preferred_element_type=f32)

    q = q_ref[...]
    kfull = k_ref[...]
    pos = pos_ref[...]
    m = m_ref[...]

    qf = dot(q, wq2_ref[...])

    ph = jnp.maximum(dot(pos, wp1b_ref[...]) + bp1a_ref[...], 0.0)
    posf_all = dot(ph, wp2c_ref[...])

    msel = dot(m, esel_ref[...])
    cnt = msel[:, NPAIR * PW:]
    all_inv = cnt == 0.0
    cnt_eff = jnp.where(all_inv, float(N_SRC), cnt)
    inv_cnt = 1.0 / cnt_eff
    inv_cm1 = 1.0 / jnp.maximum(cnt_eff - 1.0, 1.0)

    kf = [None] * NPAIR
    vf = [None] * NPAIR
    posf = [None] * NPAIR
    w = [None] * NPAIR
    for p in range(NPAIR):
        sl = slice(p * PW, (p + 1) * PW)
        kf[p] = dot(kfull[:, sl], wk2_ref[...])
        vf[p] = dot(kfull[:, sl], wv2_ref[...])
        posf[p] = posf_all[:, sl] + bp2p_ref[...]
        w[p] = jnp.where(all_inv, 1.0, msel[:, sl])

    swk = sum(kf[p] * w[p] for p in range(NPAIR))
    swk = swk + _roll64(swk)
    mean_k = swk * inv_cnt
    svar = sum(((kf[p] - mean_k) ** 2) * w[p] for p in range(NPAIR))
    svar = svar + _roll64(svar)
    sabs = sum(jnp.abs(kf[p]) * w[p] for p in range(NPAIR))
    sabs = sabs + _roll64(sabs)
    var = (svar * inv_cm1)[:, :DIM]
    single = (cnt_eff == 1.0)[:, :DIM]
    std = jnp.sqrt(jnp.where(single, 1.0, jnp.maximum(var, 0.0)))
    mean_abs = (sabs * inv_cnt)[:, :DIM]
    kstd_ref[...] = jnp.where(single, 0.0, std)
    nkstd_ref[...] = jnp.where(single, 0.0, std / (mean_abs + TINY_NUMBER))

    logits = [None] * NPAIR
    for p in range(NPAIR):
        ap = kf[p] - qf + posf[p]
        a1 = jnp.maximum(dot(ap, wa12_ref[...]) + ba1p_ref[...], 0.0)
        logits[p] = dot(a1, wa22_ref[...]) + ba2p_ref[...]

    neg = jnp.float32(-jnp.inf)
    lmax = logits[0]
    lmax = jnp.where(w[0] == 0.0, neg, lmax)
    for p in range(1, NPAIR):
        lmax = jnp.maximum(lmax, jnp.where(w[p] == 0.0, neg, logits[p]))
    lmax = jnp.maximum(lmax, _roll64(lmax))

    e = [None] * NPAIR
    for p in range(NPAIR):
        e[p] = jnp.where(w[p] == 0.0, 0.0, jnp.exp(logits[p] - lmax))
    esum = sum(e)
    esum = esum + _roll64(esum)
    inv_esum = 1.0 / esum

    xacc = jnp.zeros_like(qf)
    for p in range(NPAIR):
        attn_p = e[p] * inv_esum
        attn_ref[:, p * PW:(p + 1) * PW] = attn_p
        xacc = xacc + (vf[p] + posf[p]) * attn_p

    x_ref[...] = dot(xacc, wos_ref[...]) + bo_ref[...]


def kernel(q, k, pos, mask, Wq, Wk, Wv, Wp1, bp1, Wp2, bp2, Wa1, ba1, Wa2, ba2, Wo, bo):
    R, S, N, D = N_RAY, N_SAMPLE, N_SRC, DIM
    RS = R * S
    q2 = q.reshape(RS, D)
    k2 = k.reshape(RS, N * D)
    pos2 = pos.reshape(RS, N * 4)
    m2 = mask.reshape(RS, N).astype(jnp.float32)

    eye2 = jnp.eye(2, dtype=jnp.float32)
    Wq2 = jnp.concatenate([Wq, Wq], axis=1)
    Wk2 = jnp.kron(eye2, Wk)
    Wv2 = jnp.kron(eye2, Wv)
    Wp1b = jnp.kron(jnp.eye(N, dtype=jnp.float32), Wp1)
    bp1a = jnp.tile(bp1, N).reshape(1, N * H)
    W2c = jnp.zeros((N * H, N * D), jnp.float32)
    for n in range(N):
        W2c = W2c.at[n * H:(n + 1) * H, n * D:(n + 1) * D].set(Wp2)
    bp2p = jnp.tile(bp2, 2).reshape(1, PW)
    Wa12 = jnp.kron(eye2, Wa1)
    ba1p = jnp.tile(ba1, 2).reshape(1, 2 * H)
    Wa22 = jnp.kron(eye2, Wa2)
    ba2p = jnp.tile(ba2, 2).reshape(1, PW)
    Wos = jnp.concatenate([Wo, Wo], axis=0)
    bor = bo.reshape(1, D)
    Es = jnp.zeros((N, (NPAIR + 1) * PW), jnp.float32)
    for n in range(N):
        p, half = n // 2, n % 2
        Es = Es.at[n, p * PW + half * D:p * PW + (half + 1) * D].set(1.0)
    Es = Es.at[:, NPAIR * PW:].set(1.0)

    grid = (RS // BLK_T,)
    tok = lambda i: (i, 0)
    rep = lambda i: (0, 0)

    def wspec(w):
        return pl.BlockSpec(w.shape, rep)

    out = pl.pallas_call(
        _body,
        grid=grid,
        in_specs=[
            pl.BlockSpec((BLK_T, D), tok),
            pl.BlockSpec((BLK_T, N * D), tok),
            pl.BlockSpec((BLK_T, N * 4), tok),
            pl.BlockSpec((BLK_T, N), tok),
            wspec(Wq2), wspec(Wk2), wspec(Wv2),
            wspec(Wp1b), wspec(bp1a), wspec(W2c), wspec(bp2p),
            wspec(Wa12), wspec(ba1p), wspec(Wa22), wspec(ba2p),
            wspec(Wos), wspec(bor), wspec(Es),
        ],
        out_specs=[
            pl.BlockSpec((BLK_T, D), tok),
            pl.BlockSpec((BLK_T, N * D), tok),
            pl.BlockSpec((BLK_T, D), tok),
            pl.BlockSpec((BLK_T, D), tok),
        ],
        out_shape=[
            jax.ShapeDtypeStruct((RS, D), jnp.float32),
            jax.ShapeDtypeStruct((RS, N * D), jnp.float32),
            jax.ShapeDtypeStruct((RS, D), jnp.float32),
            jax.ShapeDtypeStruct((RS, D), jnp.float32),
        ],
    )(q2, k2, pos2, m2, Wq2, Wk2, Wv2, Wp1b, bp1a, W2c, bp2p,
      Wa12, ba1p, Wa22, ba2p, Wos, bor, Es)

    x, attn, kstd, nkstd = out
    return (x.reshape(R, S, D), attn.reshape(R, S, N, D),
            kstd.reshape(R, S, D), nkstd.reshape(R, S, D))


# speedup vs baseline: 3.3595x; 2.2547x over previous
import functools

import jax
import jax.numpy as jnp
from jax.experimental import pallas as pl

TINY_NUMBER = 1e-6
N_RAY, N_SAMPLE, N_SRC, DIM = 512, 64, 8, 64
H = DIM // 8


def _body(q_ref, k_ref, pos_ref, m_ref,
          wq_ref, wk_ref, wv_ref, wp1_ref, bp1_ref, wp2_ref, bp2_ref,
          wa1_ref, ba1_ref, wa2_ref, ba2_ref, wo_ref, bo_ref,
          x_ref, attn_ref, kstd_ref, nkstd_ref):
    R = q_ref.shape[-1]
    N = N_SRC
    dot = functools.partial(jnp.dot, preferred_element_type=jnp.float32)

    qf = dot(wq_ref[...], q_ref[0])

    m = [m_ref[0, n:n + 1, :] for n in range(N)]
    cnt = m[0]
    for n in range(1, N):
        cnt = cnt + m[n]
    all_inv = cnt == 0.0
    cnt_eff = jnp.where(all_inv, float(N), cnt)
    inv_cnt = 1.0 / cnt_eff
    inv_cm1 = 1.0 / jnp.maximum(cnt_eff - 1.0, 1.0)

    kf = [None] * N
    vf = [None] * N
    posf = [None] * N
    w = [None] * N
    for n in range(N):
        kn = k_ref[0, n]
        kf[n] = dot(wk_ref[...], kn)
        vf[n] = dot(wv_ref[...], kn)
        ph = jnp.maximum(dot(wp1_ref[...], pos_ref[0, n]) + bp1_ref[...], 0.0)
        posf[n] = dot(wp2_ref[...], ph) + bp2_ref[...]
        w[n] = jnp.where(all_inv, 1.0, m[n])

    swk = sum(kf[n] * w[n] for n in range(N))
    mean_k = swk * inv_cnt
    svar = sum(((kf[n] - mean_k) ** 2) * w[n] for n in range(N))
    sabs = sum(jnp.abs(kf[n]) * w[n] for n in range(N))
    var = svar * inv_cm1
    single = cnt_eff == 1.0
    std = jnp.sqrt(jnp.where(single, 1.0, jnp.maximum(var, 0.0)))
    mean_abs = sabs * inv_cnt
    kstd_ref[0] = jnp.where(single, 0.0, std)
    nkstd_ref[0] = jnp.where(single, 0.0, std / (mean_abs + TINY_NUMBER))

    logits = [None] * N
    for n in range(N):
        ap = kf[n] - qf + posf[n]
        a1 = jnp.maximum(dot(wa1_ref[...], ap) + ba1_ref[...], 0.0)
        logits[n] = dot(wa2_ref[...], a1) + ba2_ref[...]

    neg = jnp.float32(-jnp.inf)
    lmax = jnp.where(w[0] == 0.0, neg, logits[0])
    for n in range(1, N):
        lmax = jnp.maximum(lmax, jnp.where(w[n] == 0.0, neg, logits[n]))

    e = [jnp.where(w[n] == 0.0, 0.0, jnp.exp(logits[n] - lmax)) for n in range(N)]
    esum = sum(e)
    inv_esum = 1.0 / esum

    xacc = jnp.zeros_like(qf)
    for n in range(N):
        attn_n = e[n] * inv_esum
        attn_ref[0, n] = attn_n
        xacc = xacc + (vf[n] + posf[n]) * attn_n

    x_ref[0] = dot(wo_ref[...], xacc) + bo_ref[...]


def kernel(q, k, pos, mask, Wq, Wk, Wv, Wp1, bp1, Wp2, bp2, Wa1, ba1, Wa2, ba2, Wo, bo):
    R, S, N, D = N_RAY, N_SAMPLE, N_SRC, DIM
    f32 = jnp.float32

    qT = q.transpose(1, 2, 0)
    kT = k.transpose(1, 2, 3, 0)
    posT = pos.transpose(1, 2, 3, 0)
    mT = mask[..., 0].astype(f32).transpose(1, 2, 0)

    WqT, WkT, WvT, WoT = Wq.T, Wk.T, Wv.T, Wo.T
    Wp1T, Wp2T, Wa1T, Wa2T = Wp1.T, Wp2.T, Wa1.T, Wa2.T
    bp1t = jnp.broadcast_to(bp1[:, None], (H, R))
    bp2t = jnp.broadcast_to(bp2[:, None], (D, R))
    ba1t = jnp.broadcast_to(ba1[:, None], (H, R))
    ba2t = jnp.broadcast_to(ba2[:, None], (D, R))
    bot = jnp.broadcast_to(bo[:, None], (D, R))

    grid = (S,)

    def wspec(wshape):
        return pl.BlockSpec(wshape, lambda s: (0, 0))

    out = pl.pallas_call(
        _body,
        grid=grid,
        in_specs=[
            pl.BlockSpec((1, D, R), lambda s: (s, 0, 0)),
            pl.BlockSpec((1, N, D, R), lambda s: (s, 0, 0, 0)),
            pl.BlockSpec((1, N, 4, R), lambda s: (s, 0, 0, 0)),
            pl.BlockSpec((1, N, R), lambda s: (s, 0, 0)),
            wspec((D, D)), wspec((D, D)), wspec((D, D)),
            wspec((H, 4)), wspec((H, R)), wspec((D, H)), wspec((D, R)),
            wspec((H, D)), wspec((H, R)), wspec((D, H)), wspec((D, R)),
            wspec((D, D)), wspec((D, R)),
        ],
        out_specs=[
            pl.BlockSpec((1, D, R), lambda s: (s, 0, 0)),
            pl.BlockSpec((1, N, D, R), lambda s: (s, 0, 0, 0)),
            pl.BlockSpec((1, D, R), lambda s: (s, 0, 0)),
            pl.BlockSpec((1, D, R), lambda s: (s, 0, 0)),
        ],
        out_shape=[
            jax.ShapeDtypeStruct((S, D, R), f32),
            jax.ShapeDtypeStruct((S, N, D, R), f32),
            jax.ShapeDtypeStruct((S, D, R), f32),
            jax.ShapeDtypeStruct((S, D, R), f32),
        ],
    )(qT, kT, posT, mT, WqT, WkT, WvT, Wp1T, bp1t, Wp2T, bp2t,
      Wa1T, ba1t, Wa2T, ba2t, WoT, bot)

    xT, attnT, kstdT, nkstdT = out
    return (xT.transpose(2, 0, 1), attnT.transpose(3, 0, 1, 2),
            kstdT.transpose(2, 0, 1), nkstdT.transpose(2, 0, 1))


# speedup vs baseline: 5.6261x; 1.6747x over previous
import functools

import jax
import jax.numpy as jnp
from jax.experimental import pallas as pl
from jax.experimental.pallas import tpu as pltpu

TINY_NUMBER = 1e-6
N_RAY, N_SAMPLE, N_SRC, DIM = 512, 64, 8, 64
H = DIM // 8


def _body(q_ref, k_ref, pos_ref, m_ref,
          wq_ref, wk_ref, wv_ref, wp1_ref, bp1_ref, wp2_ref, bp2_ref,
          wa1_ref, ba1_ref, wa2_ref, ba2_ref, wo_ref, bo_ref,
          x_ref, attn_ref, kstd_ref, nkstd_ref,
          vp_ref, ph_ref):
    N = N_SRC
    dot = functools.partial(jnp.dot, preferred_element_type=jnp.float32)

    qf = dot(wq_ref[...], q_ref[0])

    m = [m_ref[0, n:n + 1, :] for n in range(N)]
    cnt = m[0]
    for n in range(1, N):
        cnt = cnt + m[n]
    all_inv = cnt == 0.0
    cnt_eff = jnp.where(all_inv, float(N), cnt)
    inv_cnt = 1.0 / cnt_eff
    inv_cm1 = 1.0 / jnp.maximum(cnt_eff - 1.0, 1.0)
    w = [jnp.where(all_inv, 1.0, m[n]) for n in range(N)]

    swk = jnp.zeros_like(qf)
    swk2 = jnp.zeros_like(qf)
    sabs = jnp.zeros_like(qf)
    for n in range(N):
        kf = dot(wk_ref[...], k_ref[0, n])
        kfw = kf * w[n]
        swk = swk + kfw
        swk2 = swk2 + kf * kfw
        sabs = sabs + jnp.abs(kfw)
        attn_ref[0, n] = kf - qf

    mean_k = swk * inv_cnt
    var = (swk2 - swk * mean_k) * inv_cm1
    single = cnt_eff == 1.0
    std = jnp.sqrt(jnp.where(single, 1.0, jnp.maximum(var, 0.0)))
    mean_abs = sabs * inv_cnt
    kstd_ref[0] = jnp.where(single, 0.0, std)
    nkstd_ref[0] = jnp.where(single, 0.0, std / (mean_abs + TINY_NUMBER))

    for n in range(N):
        vp_ref[n] = dot(wv_ref[...], k_ref[0, n])

    for n in range(N):
        ph_ref[n] = jnp.maximum(dot(wp1_ref[...], pos_ref[0, n]) + bp1_ref[...], 0.0)

    for n in range(N):
        pf = dot(wp2_ref[...], ph_ref[n]) + bp2_ref[...]
        vp_ref[n] = vp_ref[n] + pf
        attn_ref[0, n] = attn_ref[0, n] + pf

    for n in range(N):
        ph_ref[n] = jnp.maximum(dot(wa1_ref[...], attn_ref[0, n]) + ba1_ref[...], 0.0)

    for n in range(N):
        attn_ref[0, n] = dot(wa2_ref[...], ph_ref[n]) + ba2_ref[...]

    neg = jnp.float32(-jnp.inf)
    lmax = jnp.where(w[0] == 0.0, neg, attn_ref[0, 0])
    for n in range(1, N):
        lmax = jnp.maximum(lmax, jnp.where(w[n] == 0.0, neg, attn_ref[0, n]))

    esum = jnp.zeros_like(qf)
    for n in range(N):
        e = jnp.where(w[n] == 0.0, 0.0, jnp.exp(attn_ref[0, n] - lmax))
        esum = esum + e
        attn_ref[0, n] = e
    inv_esum = 1.0 / esum

    xacc = jnp.zeros_like(qf)
    for n in range(N):
        attn_n = attn_ref[0, n] * inv_esum
        attn_ref[0, n] = attn_n
        xacc = xacc + vp_ref[n] * attn_n

    x_ref[0] = dot(wo_ref[...], xacc) + bo_ref[...]


def kernel(q, k, pos, mask, Wq, Wk, Wv, Wp1, bp1, Wp2, bp2, Wa1, ba1, Wa2, ba2, Wo, bo):
    R, S, N, D = N_RAY, N_SAMPLE, N_SRC, DIM
    f32 = jnp.float32

    qT = q.transpose(1, 2, 0)
    kT = k.transpose(1, 2, 3, 0)
    posT = pos.transpose(1, 2, 3, 0)
    mT = mask[..., 0].astype(f32).transpose(1, 2, 0)

    WqT, WkT, WvT, WoT = Wq.T, Wk.T, Wv.T, Wo.T
    Wp1T, Wp2T, Wa1T, Wa2T = Wp1.T, Wp2.T, Wa1.T, Wa2.T
    bp1t = jnp.broadcast_to(bp1[:, None], (H, R))
    bp2t = jnp.broadcast_to(bp2[:, None], (D, R))
    ba1t = jnp.broadcast_to(ba1[:, None], (H, R))
    ba2t = jnp.broadcast_to(ba2[:, None], (D, R))
    bot = jnp.broadcast_to(bo[:, None], (D, R))

    grid = (S,)

    def wspec(wshape):
        return pl.BlockSpec(wshape, lambda s: (0, 0))

    out = pl.pallas_call(
        _body,
        grid=grid,
        in_specs=[
            pl.BlockSpec((1, D, R), lambda s: (s, 0, 0)),
            pl.BlockSpec((1, N, D, R), lambda s: (s, 0, 0, 0)),
            pl.BlockSpec((1, N, 4, R), lambda s: (s, 0, 0, 0)),
            pl.BlockSpec((1, N, R), lambda s: (s, 0, 0)),
            wspec((D, D)), wspec((D, D)), wspec((D, D)),
            wspec((H, 4)), wspec((H, R)), wspec((D, H)), wspec((D, R)),
            wspec((H, D)), wspec((H, R)), wspec((D, H)), wspec((D, R)),
            wspec((D, D)), wspec((D, R)),
        ],
        out_specs=[
            pl.BlockSpec((1, D, R), lambda s: (s, 0, 0)),
            pl.BlockSpec((1, N, D, R), lambda s: (s, 0, 0, 0)),
            pl.BlockSpec((1, D, R), lambda s: (s, 0, 0)),
            pl.BlockSpec((1, D, R), lambda s: (s, 0, 0)),
        ],
        out_shape=[
            jax.ShapeDtypeStruct((S, D, R), f32),
            jax.ShapeDtypeStruct((S, N, D, R), f32),
            jax.ShapeDtypeStruct((S, D, R), f32),
            jax.ShapeDtypeStruct((S, D, R), f32),
        ],
        scratch_shapes=[
            pltpu.VMEM((N, D, R), f32),
            pltpu.VMEM((N, H, R), f32),
        ],
    )(qT, kT, posT, mT, WqT, WkT, WvT, Wp1T, bp1t, Wp2T, bp2t,
      Wa1T, ba1t, Wa2T, ba2t, WoT, bot)

    xT, attnT, kstdT, nkstdT = out
    return (xT.transpose(2, 0, 1), attnT.transpose(3, 0, 1, 2),
            kstdT.transpose(2, 0, 1), nkstdT.transpose(2, 0, 1))


# speedup vs baseline: 6.2127x; 1.1043x over previous
import functools

import jax
import jax.numpy as jnp
from jax.experimental import pallas as pl
from jax.experimental.pallas import tpu as pltpu

TINY_NUMBER = 1e-6
N_RAY, N_SAMPLE, N_SRC, DIM = 512, 64, 8, 64
H = DIM // 8


def _body(q_ref, k_ref, pos_ref, m_ref,
          wqs_ref, wkvs_ref, wp1_ref, bp1_ref, wp2s_ref, bp2_ref,
          ba1c_ref, wa2_ref, ba2_ref, wo_ref, bo_ref,
          x_ref, attn_ref, kstd_ref, nkstd_ref,
          vp_ref, a1k_ref, ph_ref):
    N = N_SRC
    D = DIM
    dot = functools.partial(jnp.dot, preferred_element_type=jnp.float32)

    qq = dot(wqs_ref[...], q_ref[0])
    qf = qq[:D]
    a1q = qq[D:]

    m = [m_ref[0, n:n + 1, :] for n in range(N)]
    cnt = m[0]
    for n in range(1, N):
        cnt = cnt + m[n]
    all_inv = cnt == 0.0
    cnt_eff = jnp.where(all_inv, float(N), cnt)
    inv_cnt = 1.0 / cnt_eff
    inv_cm1 = 1.0 / jnp.maximum(cnt_eff - 1.0, 1.0)
    w = [jnp.where(all_inv, 1.0, m[n]) for n in range(N)]

    swk = jnp.zeros_like(qf)
    swk2 = jnp.zeros_like(qf)
    sabs = jnp.zeros_like(qf)
    for n in range(N):
        kv = dot(wkvs_ref[...], k_ref[0, n])
        kf = kv[:D]
        vp_ref[n] = kv[D:2 * D]
        a1k_ref[n] = kv[2 * D:]
        kfw = kf * w[n]
        swk = swk + kfw
        swk2 = swk2 + kf * kfw
        sabs = sabs + jnp.abs(kfw)

    mean_k = swk * inv_cnt
    var = (swk2 - swk * mean_k) * inv_cm1
    single = cnt_eff == 1.0
    std = jnp.sqrt(jnp.where(single, 1.0, jnp.maximum(var, 0.0)))
    mean_abs = sabs * inv_cnt
    kstd_ref[0] = jnp.where(single, 0.0, std)
    nkstd_ref[0] = jnp.where(single, 0.0, std / (mean_abs + TINY_NUMBER))

    for n in range(N):
        ph_ref[n] = jnp.maximum(dot(wp1_ref[...], pos_ref[0, n]) + bp1_ref[...], 0.0)

    for n in range(N):
        pp = dot(wp2s_ref[...], ph_ref[n])
        vp_ref[n] = vp_ref[n] + pp[:D] + bp2_ref[...]
        a1k_ref[n] = jnp.maximum(a1k_ref[n] - a1q + pp[D:] + ba1c_ref[...], 0.0)

    for n in range(N):
        attn_ref[0, n] = dot(wa2_ref[...], a1k_ref[n]) + ba2_ref[...]

    neg = jnp.float32(-jnp.inf)
    lmax = jnp.where(w[0] == 0.0, neg, attn_ref[0, 0])
    for n in range(1, N):
        lmax = jnp.maximum(lmax, jnp.where(w[n] == 0.0, neg, attn_ref[0, n]))

    esum = jnp.zeros_like(qf)
    xe = jnp.zeros_like(qf)
    for n in range(N):
        e = jnp.where(w[n] == 0.0, 0.0, jnp.exp(attn_ref[0, n] - lmax))
        esum = esum + e
        xe = xe + vp_ref[n] * e
        attn_ref[0, n] = e
    inv_esum = 1.0 / esum

    for n in range(N):
        attn_ref[0, n] = attn_ref[0, n] * inv_esum

    x_ref[0] = dot(wo_ref[...], xe * inv_esum) + bo_ref[...]


def kernel(q, k, pos, mask, Wq, Wk, Wv, Wp1, bp1, Wp2, bp2, Wa1, ba1, Wa2, ba2, Wo, bo):
    R, S, N, D = N_RAY, N_SAMPLE, N_SRC, DIM
    f32 = jnp.float32

    qT = q.transpose(1, 2, 0)
    kT = k.transpose(1, 2, 3, 0)
    posT = pos.transpose(1, 2, 3, 0)
    mT = mask[..., 0].astype(f32).transpose(1, 2, 0)

    Wa1k = Wa1.T @ Wk.T
    Wa1q = Wa1.T @ Wq.T
    Wa1p = Wa1.T @ Wp2.T
    Wqs = jnp.concatenate([Wq.T, Wa1q], axis=0)
    Wkvs = jnp.concatenate([Wk.T, Wv.T, Wa1k], axis=0)
    Wp2s = jnp.concatenate([Wp2.T, Wa1p], axis=0)
    ba1c = Wa1.T @ bp2 + ba1

    bp1t = jnp.broadcast_to(bp1[:, None], (H, R))
    bp2t = jnp.broadcast_to(bp2[:, None], (D, R))
    ba1t = jnp.broadcast_to(ba1c[:, None], (H, R))
    ba2t = jnp.broadcast_to(ba2[:, None], (D, R))
    bot = jnp.broadcast_to(bo[:, None], (D, R))

    grid = (S,)

    def wspec(wshape):
        return pl.BlockSpec(wshape, lambda s: (0, 0))

    out = pl.pallas_call(
        _body,
        grid=grid,
        in_specs=[
            pl.BlockSpec((1, D, R), lambda s: (s, 0, 0)),
            pl.BlockSpec((1, N, D, R), lambda s: (s, 0, 0, 0)),
            pl.BlockSpec((1, N, 4, R), lambda s: (s, 0, 0, 0)),
            pl.BlockSpec((1, N, R), lambda s: (s, 0, 0)),
            wspec((D + H, D)), wspec((2 * D + H, D)),
            wspec((H, 4)), wspec((H, R)), wspec((D + H, H)), wspec((D, R)),
            wspec((H, R)), wspec((D, H)), wspec((D, R)),
            wspec((D, D)), wspec((D, R)),
        ],
        out_specs=[
            pl.BlockSpec((1, D, R), lambda s: (s, 0, 0)),
            pl.BlockSpec((1, N, D, R), lambda s: (s, 0, 0, 0)),
            pl.BlockSpec((1, D, R), lambda s: (s, 0, 0)),
            pl.BlockSpec((1, D, R), lambda s: (s, 0, 0)),
        ],
        out_shape=[
            jax.ShapeDtypeStruct((S, D, R), f32),
            jax.ShapeDtypeStruct((S, N, D, R), f32),
            jax.ShapeDtypeStruct((S, D, R), f32),
            jax.ShapeDtypeStruct((S, D, R), f32),
        ],
        scratch_shapes=[
            pltpu.VMEM((N, D, R), f32),
            pltpu.VMEM((N, H, R), f32),
            pltpu.VMEM((N, H, R), f32),
        ],
    )(qT, kT, posT, mT, Wqs, Wkvs, Wp1.T, bp1t, Wp2s, bp2t,
      ba1t, Wa2.T, ba2t, Wo.T, bot)

    xT, attnT, kstdT, nkstdT = out
    return (xT.transpose(2, 0, 1), attnT.transpose(3, 0, 1, 2),
            kstdT.transpose(2, 0, 1), nkstdT.transpose(2, 0, 1))


# speedup vs baseline: 6.8316x; 1.0996x over previous
import functools

import jax
import jax.numpy as jnp
from jax.experimental import pallas as pl
from jax.experimental.pallas import tpu as pltpu

TINY_NUMBER = 1e-6
N_RAY, N_SAMPLE, N_SRC, DIM = 512, 64, 8, 64
H = DIM // 8
SB = 2


def _body(q_ref, k_ref, pos_ref, m_ref,
          wqs_ref, wkvs_ref, wp1_ref, bp1_ref, wp2s_ref, bp2_ref,
          ba1c_ref, wa2_ref, ba2_ref, wo_ref, bo_ref,
          x_ref, attn_ref, kstd_ref, nkstd_ref,
          vp_ref, a1k_ref, ph_ref):
    for s in range(SB):
        _sample(s, q_ref, k_ref, pos_ref, m_ref,
                wqs_ref, wkvs_ref, wp1_ref, bp1_ref, wp2s_ref, bp2_ref,
                ba1c_ref, wa2_ref, ba2_ref, wo_ref, bo_ref,
                x_ref, attn_ref, kstd_ref, nkstd_ref,
                vp_ref, a1k_ref, ph_ref)


def _sample(s, q_ref, k_ref, pos_ref, m_ref,
            wqs_ref, wkvs_ref, wp1_ref, bp1_ref, wp2s_ref, bp2_ref,
            ba1c_ref, wa2_ref, ba2_ref, wo_ref, bo_ref,
            x_ref, attn_ref, kstd_ref, nkstd_ref,
            vp_ref, a1k_ref, ph_ref):
    N = N_SRC
    D = DIM
    dot = functools.partial(jnp.dot, preferred_element_type=jnp.float32)

    qq = dot(wqs_ref[...], q_ref[s])
    qf = qq[:D]
    a1q = qq[D:]

    m = [m_ref[s, n:n + 1, :] for n in range(N)]
    cnt = m[0]
    for n in range(1, N):
        cnt = cnt + m[n]
    all_inv = cnt == 0.0
    cnt_eff = jnp.where(all_inv, float(N), cnt)
    inv_cnt = 1.0 / cnt_eff
    inv_cm1 = 1.0 / jnp.maximum(cnt_eff - 1.0, 1.0)
    w = [jnp.where(all_inv, 1.0, m[n]) for n in range(N)]

    swk = jnp.zeros_like(qf)
    swk2 = jnp.zeros_like(qf)
    sabs = jnp.zeros_like(qf)
    for n in range(N):
        kv = dot(wkvs_ref[...], k_ref[s, n])
        kf = kv[:D]
        vp_ref[n] = kv[D:2 * D]
        a1k_ref[n] = kv[2 * D:]
        kfw = kf * w[n]
        swk = swk + kfw
        swk2 = swk2 + kf * kfw
        sabs = sabs + jnp.abs(kfw)

    mean_k = swk * inv_cnt
    var = (swk2 - swk * mean_k) * inv_cm1
    single = cnt_eff == 1.0
    std = jnp.sqrt(jnp.where(single, 1.0, jnp.maximum(var, 0.0)))
    mean_abs = sabs * inv_cnt
    kstd_ref[s] = jnp.where(single, 0.0, std)
    nkstd_ref[s] = jnp.where(single, 0.0, std / (mean_abs + TINY_NUMBER))

    for n in range(N):
        ph_ref[n] = jnp.maximum(dot(wp1_ref[...], pos_ref[s, n]) + bp1_ref[...], 0.0)

    for n in range(N):
        pp = dot(wp2s_ref[...], ph_ref[n])
        vp_ref[n] = vp_ref[n] + pp[:D] + bp2_ref[...]
        a1k_ref[n] = jnp.maximum(a1k_ref[n] - a1q + pp[D:] + ba1c_ref[...], 0.0)

    for n in range(N):
        attn_ref[s, n] = dot(wa2_ref[...], a1k_ref[n]) + ba2_ref[...]

    neg = jnp.float32(-jnp.inf)
    lmax = jnp.where(w[0] == 0.0, neg, attn_ref[s, 0])
    for n in range(1, N):
        lmax = jnp.maximum(lmax, jnp.where(w[n] == 0.0, neg, attn_ref[s, n]))

    esum = jnp.zeros_like(qf)
    xe = jnp.zeros_like(qf)
    for n in range(N):
        e = jnp.where(w[n] == 0.0, 0.0, jnp.exp(attn_ref[s, n] - lmax))
        esum = esum + e
        xe = xe + vp_ref[n] * e
        attn_ref[s, n] = e
    inv_esum = 1.0 / esum

    for n in range(N):
        attn_ref[s, n] = attn_ref[s, n] * inv_esum

    x_ref[s] = dot(wo_ref[...], xe * inv_esum) + bo_ref[...]


def kernel(q, k, pos, mask, Wq, Wk, Wv, Wp1, bp1, Wp2, bp2, Wa1, ba1, Wa2, ba2, Wo, bo):
    R, S, N, D = N_RAY, N_SAMPLE, N_SRC, DIM
    f32 = jnp.float32

    qT = q.transpose(1, 2, 0)
    kT = k.transpose(1, 2, 3, 0)
    posT = pos.transpose(1, 2, 3, 0)
    mT = mask[..., 0].astype(f32).transpose(1, 2, 0)

    Wa1k = Wa1.T @ Wk.T
    Wa1q = Wa1.T @ Wq.T
    Wa1p = Wa1.T @ Wp2.T
    Wqs = jnp.concatenate([Wq.T, Wa1q], axis=0)
    Wkvs = jnp.concatenate([Wk.T, Wv.T, Wa1k], axis=0)
    Wp2s = jnp.concatenate([Wp2.T, Wa1p], axis=0)
    ba1c = Wa1.T @ bp2 + ba1

    bp1t = jnp.broadcast_to(bp1[:, None], (H, R))
    bp2t = jnp.broadcast_to(bp2[:, None], (D, R))
    ba1t = jnp.broadcast_to(ba1c[:, None], (H, R))
    ba2t = jnp.broadcast_to(ba2[:, None], (D, R))
    bot = jnp.broadcast_to(bo[:, None], (D, R))

    grid = (S // SB,)

    def wspec(wshape):
        return pl.BlockSpec(wshape, lambda s: (0, 0))

    out = pl.pallas_call(
        _body,
        grid=grid,
        in_specs=[
            pl.BlockSpec((SB, D, R), lambda s: (s, 0, 0)),
            pl.BlockSpec((SB, N, D, R), lambda s: (s, 0, 0, 0)),
            pl.BlockSpec((SB, N, 4, R), lambda s: (s, 0, 0, 0)),
            pl.BlockSpec((SB, N, R), lambda s: (s, 0, 0)),
            wspec((D + H, D)), wspec((2 * D + H, D)),
            wspec((H, 4)), wspec((H, R)), wspec((D + H, H)), wspec((D, R)),
            wspec((H, R)), wspec((D, H)), wspec((D, R)),
            wspec((D, D)), wspec((D, R)),
        ],
        out_specs=[
            pl.BlockSpec((SB, D, R), lambda s: (s, 0, 0)),
            pl.BlockSpec((SB, N, D, R), lambda s: (s, 0, 0, 0)),
            pl.BlockSpec((SB, D, R), lambda s: (s, 0, 0)),
            pl.BlockSpec((SB, D, R), lambda s: (s, 0, 0)),
        ],
        out_shape=[
            jax.ShapeDtypeStruct((S, D, R), f32),
            jax.ShapeDtypeStruct((S, N, D, R), f32),
            jax.ShapeDtypeStruct((S, D, R), f32),
            jax.ShapeDtypeStruct((S, D, R), f32),
        ],
        scratch_shapes=[
            pltpu.VMEM((N, D, R), f32),
            pltpu.VMEM((N, H, R), f32),
            pltpu.VMEM((N, H, R), f32),
        ],
    )(qT, kT, posT, mT, Wqs, Wkvs, Wp1.T, bp1t, Wp2s, bp2t,
      ba1t, Wa2.T, ba2t, Wo.T, bot)

    xT, attnT, kstdT, nkstdT = out
    return (xT.transpose(2, 0, 1), attnT.transpose(3, 0, 1, 2),
            kstdT.transpose(2, 0, 1), nkstdT.transpose(2, 0, 1))


# speedup vs baseline: 6.9331x; 1.0148x over previous
import functools

import jax
import jax.numpy as jnp
from jax.experimental import pallas as pl
from jax.experimental.pallas import tpu as pltpu

TINY_NUMBER = 1e-6
N_RAY, N_SAMPLE, N_SRC, DIM = 512, 64, 8, 64
H = DIM // 8
SB = 4


def _body(q_ref, k_ref, pos_ref, m_ref,
          wqs_ref, wkvs_ref, wp1_ref, bp1_ref, wp2s_ref, bp2_ref,
          ba1c_ref, wa2_ref, ba2_ref, wo_ref, bo_ref,
          x_ref, attn_ref, kstd_ref, nkstd_ref,
          vp_ref, a1k_ref, ph_ref):
    for s in range(SB):
        _sample(s, q_ref, k_ref, pos_ref, m_ref,
                wqs_ref, wkvs_ref, wp1_ref, bp1_ref, wp2s_ref, bp2_ref,
                ba1c_ref, wa2_ref, ba2_ref, wo_ref, bo_ref,
                x_ref, attn_ref, kstd_ref, nkstd_ref,
                vp_ref, a1k_ref, ph_ref)


def _sample(s, q_ref, k_ref, pos_ref, m_ref,
            wqs_ref, wkvs_ref, wp1_ref, bp1_ref, wp2s_ref, bp2_ref,
            ba1c_ref, wa2_ref, ba2_ref, wo_ref, bo_ref,
            x_ref, attn_ref, kstd_ref, nkstd_ref,
            vp_ref, a1k_ref, ph_ref):
    N = N_SRC
    D = DIM
    dot = functools.partial(jnp.dot, preferred_element_type=jnp.float32)

    qq = dot(wqs_ref[...], q_ref[s])
    qf = qq[:D]
    a1q = qq[D:]

    m = [m_ref[s, n:n + 1, :] for n in range(N)]
    cnt = m[0]
    for n in range(1, N):
        cnt = cnt + m[n]
    all_inv = cnt == 0.0
    cnt_eff = jnp.where(all_inv, float(N), cnt)
    inv_cnt = 1.0 / cnt_eff
    inv_cm1 = 1.0 / jnp.maximum(cnt_eff - 1.0, 1.0)
    w = [jnp.where(all_inv, 1.0, m[n]) for n in range(N)]

    swk = jnp.zeros_like(qf)
    swk2 = jnp.zeros_like(qf)
    sabs = jnp.zeros_like(qf)
    for n in range(N):
        kv = dot(wkvs_ref[...], k_ref[s, n])
        kf = kv[:D]
        vp_ref[n] = kv[D:2 * D]
        a1k_ref[n] = kv[2 * D:]
        kfw = kf * w[n]
        swk = swk + kfw
        swk2 = swk2 + kf * kfw
        sabs = sabs + jnp.abs(kfw)

    mean_k = swk * inv_cnt
    var = (swk2 - swk * mean_k) * inv_cm1
    single = cnt_eff == 1.0
    std = jnp.sqrt(jnp.where(single, 1.0, jnp.maximum(var, 0.0)))
    mean_abs = sabs * inv_cnt
    kstd_ref[s] = jnp.where(single, 0.0, std)
    nkstd_ref[s] = jnp.where(single, 0.0, std / (mean_abs + TINY_NUMBER))

    for n in range(N):
        ph_ref[n] = jnp.maximum(dot(wp1_ref[...], pos_ref[s, n]) + bp1_ref[...], 0.0)

    for n in range(N):
        pp = dot(wp2s_ref[...], ph_ref[n])
        vp_ref[n] = vp_ref[n] + pp[:D] + bp2_ref[...]
        a1k_ref[n] = jnp.maximum(a1k_ref[n] - a1q + pp[D:] + ba1c_ref[...], 0.0)

    for n in range(N):
        attn_ref[s, n] = dot(wa2_ref[...], a1k_ref[n]) + ba2_ref[...]

    neg = jnp.float32(-jnp.inf)
    lmax = jnp.where(w[0] == 0.0, neg, attn_ref[s, 0])
    for n in range(1, N):
        lmax = jnp.maximum(lmax, jnp.where(w[n] == 0.0, neg, attn_ref[s, n]))

    esum = jnp.zeros_like(qf)
    xe = jnp.zeros_like(qf)
    for n in range(N):
        e = jnp.where(w[n] == 0.0, 0.0, jnp.exp(attn_ref[s, n] - lmax))
        esum = esum + e
        xe = xe + vp_ref[n] * e
        attn_ref[s, n] = e
    inv_esum = 1.0 / esum

    for n in range(N):
        attn_ref[s, n] = attn_ref[s, n] * inv_esum

    x_ref[s] = dot(wo_ref[...], xe * inv_esum) + bo_ref[...]


def kernel(q, k, pos, mask, Wq, Wk, Wv, Wp1, bp1, Wp2, bp2, Wa1, ba1, Wa2, ba2, Wo, bo):
    R, S, N, D = N_RAY, N_SAMPLE, N_SRC, DIM
    f32 = jnp.float32

    qT = q.transpose(1, 2, 0)
    kT = k.transpose(1, 2, 3, 0)
    posT = pos.transpose(1, 2, 3, 0)
    mT = mask[..., 0].astype(f32).transpose(1, 2, 0)

    Wa1k = Wa1.T @ Wk.T
    Wa1q = Wa1.T @ Wq.T
    Wa1p = Wa1.T @ Wp2.T
    Wqs = jnp.concatenate([Wq.T, Wa1q], axis=0)
    Wkvs = jnp.concatenate([Wk.T, Wv.T, Wa1k], axis=0)
    Wp2s = jnp.concatenate([Wp2.T, Wa1p], axis=0)
    ba1c = Wa1.T @ bp2 + ba1

    bp1t = jnp.broadcast_to(bp1[:, None], (H, R))
    bp2t = jnp.broadcast_to(bp2[:, None], (D, R))
    ba1t = jnp.broadcast_to(ba1c[:, None], (H, R))
    ba2t = jnp.broadcast_to(ba2[:, None], (D, R))
    bot = jnp.broadcast_to(bo[:, None], (D, R))

    grid = (S // SB,)

    def wspec(wshape):
        return pl.BlockSpec(wshape, lambda s: (0, 0))

    out = pl.pallas_call(
        _body,
        grid=grid,
        in_specs=[
            pl.BlockSpec((SB, D, R), lambda s: (s, 0, 0)),
            pl.BlockSpec((SB, N, D, R), lambda s: (s, 0, 0, 0)),
            pl.BlockSpec((SB, N, 4, R), lambda s: (s, 0, 0, 0)),
            pl.BlockSpec((SB, N, R), lambda s: (s, 0, 0)),
            wspec((D + H, D)), wspec((2 * D + H, D)),
            wspec((H, 4)), wspec((H, R)), wspec((D + H, H)), wspec((D, R)),
            wspec((H, R)), wspec((D, H)), wspec((D, R)),
            wspec((D, D)), wspec((D, R)),
        ],
        out_specs=[
            pl.BlockSpec((SB, D, R), lambda s: (s, 0, 0)),
            pl.BlockSpec((SB, N, D, R), lambda s: (s, 0, 0, 0)),
            pl.BlockSpec((SB, D, R), lambda s: (s, 0, 0)),
            pl.BlockSpec((SB, D, R), lambda s: (s, 0, 0)),
        ],
        out_shape=[
            jax.ShapeDtypeStruct((S, D, R), f32),
            jax.ShapeDtypeStruct((S, N, D, R), f32),
            jax.ShapeDtypeStruct((S, D, R), f32),
            jax.ShapeDtypeStruct((S, D, R), f32),
        ],
        scratch_shapes=[
            pltpu.VMEM((N, D, R), f32),
            pltpu.VMEM((N, H, R), f32),
            pltpu.VMEM((N, H, R), f32),
        ],
    )(qT, kT, posT, mT, Wqs, Wkvs, Wp1.T, bp1t, Wp2s, bp2t,
      ba1t, Wa2.T, ba2t, Wo.T, bot)

    xT, attnT, kstdT, nkstdT = out
    return (xT.transpose(2, 0, 1), attnT.transpose(3, 0, 1, 2),
            kstdT.transpose(2, 0, 1), nkstdT.transpose(2, 0, 1))
